# baseline (speedup 1.0000x reference)
import jax
import jax.numpy as jnp
from jax import lax
from jax.experimental import pallas as pl
from jax.experimental.pallas import tpu as pltpu
from jax.experimental.pallas import tpu_sc as plsc

B = 16384
D = 64
C = 100000
NS = 16
SPT = B // NS
SB = 64
NSB = SPT // SB
HIST_N = 102400
TR = 25088
DUMMY = 25080
CHUNK = 25000
LW = 0.005


def _body(y_hbm, feat_hbm, centers_hbm, grad_hbm, loss_hbm,
          hist, table, lsum,
          y2d, hvm, scale2d, idx, feat_st, g_st, val_sb,
          zb1, zb2, ones, zidx, zf32, lread, lout, sem):
    c = lax.axis_index("c")
    s = lax.axis_index("s")
    lo_a = c * (2 * CHUNK)

    def zf1(k, carry):
        zb1[pl.ds(k * 16, 16)] = jnp.zeros((16,), jnp.int32)
        return carry
    lax.fori_loop(0, 800 // 16, zf1, 0)

    def zf2(t, carry):
        r = t // 4
        q = (t % 4) * 16
        zb2[r, pl.ds(q, 16)] = jnp.zeros((16,), jnp.float32)
        return carry
    lax.fori_loop(0, 32 * 4, zf2, 0)

    def of(k, carry):
        ones[pl.ds(k * 16, 16)] = jnp.ones((16,), jnp.int32)
        return carry
    lax.fori_loop(0, SB // 16, of, 0)

    zidx[pl.ds(0, 16)] = jnp.zeros((16,), jnp.int32)
    zf32[pl.ds(0, 16)] = jnp.zeros((16,), jnp.float32)

    @pl.when(s == 0)
    def _():
        pltpu.sync_copy(zf32, lsum)

    for k in range(8):
        pltpu.sync_copy(zb1, hist.at[pl.ds(s * 6400 + k * 800, 800)])

    pltpu.sync_copy(y_hbm.at[pl.ds(s * NSB, NSB), :], y2d)

    plsc.subcore_barrier()

    for j in range(NSB):
        pltpu.sync_copy(ones, hist.at[y2d.at[j]], add=True)

    plsc.subcore_barrier()

    for j in range(NSB):
        pltpu.sync_copy(hist.at[y2d.at[j]], hvm.at[j])

    def fcomp(t, carry):
        j = t // 4
        q = (t % 4) * 16
        hv = hvm[j, pl.ds(q, 16)]
        scale2d[j, pl.ds(q, 16)] = 1.0 / (1.0 + hv.astype(jnp.float32))
        return carry
    lax.fori_loop(0, (NSB * SB) // 16, fcomp, 0)

    lacc = jnp.zeros((16,), jnp.float32)
    for phase in range(2):
        lo = lo_a + phase * CHUNK

        for k in range(49):
            pltpu.sync_copy(zb2, table.at[pl.ds(s * 1568 + k * 32, 32), :])

        def icomp(t, carry):
            j = t // 4
            q = (t % 4) * 16
            yv = y2d[j, pl.ds(q, 16)]
            inc = (yv >= lo) & (yv < lo + CHUNK)
            idx[j, pl.ds(q, 16)] = jnp.where(inc, yv - lo, DUMMY)
            return carry
        lax.fori_loop(0, (NSB * SB) // 16, icomp, 0)

        plsc.subcore_barrier()

        for j in range(NSB):
            cp = pltpu.async_copy(centers_hbm.at[y2d.at[j]], g_st, sem)
            pltpu.sync_copy(feat_hbm.at[pl.ds(s * SPT + j * SB, SB), :],
                            feat_st)
            cp.wait()

            def samp(i, acc):
                g16 = (i // 16) * 16
                qv = scale2d[j, pl.ds(g16, 16)]
                scv = lax.gather(
                    qv, jnp.full((16, 1), i - g16, jnp.int32),
                    dimension_numbers=lax.GatherDimensionNumbers(
                        offset_dims=(), collapsed_slice_dims=(0,),
                        start_index_map=(0,)),
                    slice_sizes=(1,),
                    mode=lax.GatherScatterMode.PROMISE_IN_BOUNDS)
                for qq in range(D // 16):
                    fv = feat_st[i, pl.ds(qq * 16, 16)]
                    gv = g_st[i, pl.ds(qq * 16, 16)]
                    dv = fv - gv
                    if phase == 0:
                        acc = acc + dv * dv
                    val_sb[i, pl.ds(qq * 16, 16)] = (gv - fv) * scv
                return acc
            lacc = lax.fori_loop(0, SB, samp, lacc)

            pltpu.sync_copy(val_sb, table.at[idx.at[j]], add=True)

        if phase == 0:
            lout[pl.ds(0, 16)] = lacc
            pltpu.sync_copy(lout, lsum.at[zidx], add=True)

        plsc.subcore_barrier()

        @pl.when(s < 15)
        def _():
            pltpu.sync_copy(table.at[pl.ds(s * 1568, 1568), :],
                            grad_hbm.at[pl.ds(lo + s * 1568, 1568), :])

        @pl.when(s == 15)
        def _():
            pltpu.sync_copy(table.at[pl.ds(23520, 1480), :],
                            grad_hbm.at[pl.ds(lo + 23520, 1480), :])

        if phase == 0:
            @pl.when((c == 0) & (s == 0))
            def _():
                pltpu.sync_copy(lsum, lread)
                lout[pl.ds(0, 16)] = lread[pl.ds(0, 16)] * LW
                pltpu.sync_copy(lout, loss_hbm)

        plsc.subcore_barrier()


_sc_call = pl.kernel(
    _body,
    out_type=(
        jax.ShapeDtypeStruct((C, D), jnp.float32),
        jax.ShapeDtypeStruct((16,), jnp.float32),
    ),
    mesh=plsc.VectorSubcoreMesh(core_axis_name="c", subcore_axis_name="s"),
    compiler_params=pltpu.CompilerParams(use_tc_tiling_on_sc=False),
    scratch_types=[
        pltpu.VMEM_SHARED((HIST_N,), jnp.int32),
        pltpu.VMEM_SHARED((TR, D), jnp.float32),
        pltpu.VMEM_SHARED((16,), jnp.float32),
        pltpu.VMEM((NSB, SB), jnp.int32),
        pltpu.VMEM((NSB, SB), jnp.int32),
        pltpu.VMEM((NSB, SB), jnp.float32),
        pltpu.VMEM((NSB, SB), jnp.int32),
        pltpu.VMEM((SB, D), jnp.float32),
        pltpu.VMEM((SB, D), jnp.float32),
        pltpu.VMEM((SB, D), jnp.float32),
        pltpu.VMEM((800,), jnp.int32),
        pltpu.VMEM((32, D), jnp.float32),
        pltpu.VMEM((SB,), jnp.int32),
        pltpu.VMEM((16,), jnp.int32),
        pltpu.VMEM((16,), jnp.float32),
        pltpu.VMEM((16,), jnp.float32),
        pltpu.VMEM((16,), jnp.float32),
        pltpu.SemaphoreType.DMA,
    ],
)


def kernel(y, feat, centers):
    y2 = y.reshape(B // SB, SB)
    grad, lossv = _sc_call(y2, feat, centers)
    return lossv[0], grad


# speedup vs baseline: 1.0215x; 1.0215x over previous
import jax
import jax.numpy as jnp
from jax import lax
from jax.experimental import pallas as pl
from jax.experimental.pallas import tpu as pltpu
from jax.experimental.pallas import tpu_sc as plsc

B = 16384
D = 64
C = 100000
NS = 16
SPT = B // NS
SB = 64
NSB = SPT // SB
HIST_N = 102400
TR = 25088
DUMMY = 25000
CHUNK = 25000
LW = 0.005


def _body(y_hbm, feat_hbm, centers_hbm, grad_hbm, loss_hbm,
          hist, table, lsum,
          y2d, hvm, scale2d, idx, feat_st, g_st, val_sb,
          zb1, zb2, ones, zidx, zf32, lread, lout, sem, zsem):
    c = lax.axis_index("c")
    s = lax.axis_index("s")
    lo_a = c * (2 * CHUNK)

    def zf1(k, carry):
        zb1[pl.ds(k * 16, 16)] = jnp.zeros((16,), jnp.int32)
        return carry
    lax.fori_loop(0, 800 // 16, zf1, 0)

    def zf2(t, carry):
        r = t // 4
        q = (t % 4) * 16
        zb2[r, pl.ds(q, 16)] = jnp.zeros((16,), jnp.float32)
        return carry
    lax.fori_loop(0, 32 * 4, zf2, 0)

    def of(k, carry):
        ones[pl.ds(k * 16, 16)] = jnp.ones((16,), jnp.int32)
        return carry
    lax.fori_loop(0, SB // 16, of, 0)

    zidx[pl.ds(0, 16)] = jnp.zeros((16,), jnp.int32)
    zf32[pl.ds(0, 16)] = jnp.zeros((16,), jnp.float32)

    @pl.when(s == 0)
    def _():
        pltpu.sync_copy(zf32, lsum)

    hcps = [pltpu.async_copy(zb1, hist.at[pl.ds(s * 6400 + k * 800, 800)],
                             zsem) for k in range(8)]
    for cp2 in hcps:
        cp2.wait()

    pltpu.sync_copy(y_hbm.at[pl.ds(s * NSB, NSB), :], y2d)

    plsc.subcore_barrier()

    for j in range(NSB):
        pltpu.sync_copy(ones, hist.at[y2d.at[j]], add=True)

    plsc.subcore_barrier()

    for j in range(NSB):
        pltpu.sync_copy(hist.at[y2d.at[j]], hvm.at[j])

    def fcomp(t, carry):
        j = t // 4
        q = (t % 4) * 16
        hv = hvm[j, pl.ds(q, 16)]
        scale2d[j, pl.ds(q, 16)] = 1.0 / (1.0 + hv.astype(jnp.float32))
        return carry
    lax.fori_loop(0, (NSB * SB) // 16, fcomp, 0)

    lacc = jnp.zeros((16,), jnp.float32)
    for phase in range(2):
        lo = lo_a + phase * CHUNK

        zcps = [pltpu.async_copy(zb2, table.at[pl.ds(s * 1568 + k * 32, 32), :],
                                 zsem) for k in range(49)]
        for cp2 in zcps:
            cp2.wait()

        def icomp(t, carry):
            j = t // 4
            q = (t % 4) * 16
            yv = y2d[j, pl.ds(q, 16)]
            inc = (yv >= lo) & (yv < lo + CHUNK)
            dum = DUMMY + (yv & 63)
            idx[j, pl.ds(q, 16)] = jnp.where(inc, yv - lo, dum)
            return carry
        lax.fori_loop(0, (NSB * SB) // 16, icomp, 0)

        plsc.subcore_barrier()

        for j in range(NSB):
            cp = pltpu.async_copy(centers_hbm.at[y2d.at[j]], g_st, sem)
            pltpu.sync_copy(feat_hbm.at[pl.ds(s * SPT + j * SB, SB), :],
                            feat_st)
            cp.wait()

            def samp(i, acc):
                g16 = (i // 16) * 16
                qv = scale2d[j, pl.ds(g16, 16)]
                scv = lax.gather(
                    qv, jnp.full((16, 1), i - g16, jnp.int32),
                    dimension_numbers=lax.GatherDimensionNumbers(
                        offset_dims=(), collapsed_slice_dims=(0,),
                        start_index_map=(0,)),
                    slice_sizes=(1,),
                    mode=lax.GatherScatterMode.PROMISE_IN_BOUNDS)
                for qq in range(D // 16):
                    fv = feat_st[i, pl.ds(qq * 16, 16)]
                    gv = g_st[i, pl.ds(qq * 16, 16)]
                    dv = fv - gv
                    if phase == 0:
                        acc = acc + dv * dv
                    val_sb[i, pl.ds(qq * 16, 16)] = (gv - fv) * scv
                return acc
            lacc = lax.fori_loop(0, SB, samp, lacc)

            pltpu.sync_copy(val_sb, table.at[idx.at[j]], add=True)

        if phase == 0:
            lout[pl.ds(0, 16)] = lacc
            pltpu.sync_copy(lout, lsum.at[zidx], add=True)

        plsc.subcore_barrier()

        @pl.when(s < 15)
        def _():
            pltpu.sync_copy(table.at[pl.ds(s * 1568, 1568), :],
                            grad_hbm.at[pl.ds(lo + s * 1568, 1568), :])

        @pl.when(s == 15)
        def _():
            pltpu.sync_copy(table.at[pl.ds(23520, 1480), :],
                            grad_hbm.at[pl.ds(lo + 23520, 1480), :])

        if phase == 0:
            @pl.when((c == 0) & (s == 0))
            def _():
                pltpu.sync_copy(lsum, lread)
                lout[pl.ds(0, 16)] = lread[pl.ds(0, 16)] * LW
                pltpu.sync_copy(lout, loss_hbm)

        plsc.subcore_barrier()


_sc_call = pl.kernel(
    _body,
    out_type=(
        jax.ShapeDtypeStruct((C, D), jnp.float32),
        jax.ShapeDtypeStruct((16,), jnp.float32),
    ),
    mesh=plsc.VectorSubcoreMesh(core_axis_name="c", subcore_axis_name="s"),
    compiler_params=pltpu.CompilerParams(use_tc_tiling_on_sc=False),
    scratch_types=[
        pltpu.VMEM_SHARED((HIST_N,), jnp.int32),
        pltpu.VMEM_SHARED((TR, D), jnp.float32),
        pltpu.VMEM_SHARED((16,), jnp.float32),
        pltpu.VMEM((NSB, SB), jnp.int32),
        pltpu.VMEM((NSB, SB), jnp.int32),
        pltpu.VMEM((NSB, SB), jnp.float32),
        pltpu.VMEM((NSB, SB), jnp.int32),
        pltpu.VMEM((SB, D), jnp.float32),
        pltpu.VMEM((SB, D), jnp.float32),
        pltpu.VMEM((SB, D), jnp.float32),
        pltpu.VMEM((800,), jnp.int32),
        pltpu.VMEM((32, D), jnp.float32),
        pltpu.VMEM((SB,), jnp.int32),
        pltpu.VMEM((16,), jnp.int32),
        pltpu.VMEM((16,), jnp.float32),
        pltpu.VMEM((16,), jnp.float32),
        pltpu.VMEM((16,), jnp.float32),
        pltpu.SemaphoreType.DMA,
        pltpu.SemaphoreType.DMA,
    ],
)


def kernel(y, feat, centers):
    y2 = y.reshape(B // SB, SB)
    grad, lossv = _sc_call(y2, feat, centers)
    return lossv[0], grad
